# baseline (speedup 1.0000x reference)
import jax
import jax.numpy as jnp
from jax import lax
from jax.experimental import pallas as pl
from jax.experimental.pallas import tpu as pltpu
from jax.experimental.pallas import tpu_sc as plsc

N_CORES = 2
N_SUBCORES = 16
L = 16

B = 16384
F = 100
M = 49
D = 32

FH = F // 2
BW = B // N_SUBCORES
NB = 8
NP = NB * FH
NCHUNK = BW // NB


def _splat(s):
    return lax.broadcast_in_dim(s, (L,), ())


def _body(x_hbm, tab_hbm, e0_hbm, hinv_hbm, out_hbm,
          tab_v, xb, e0_v, hinv_v, e0p, hip, fbase, rowp, colg, offb, tb, outb):
    cid = lax.axis_index("c")
    sid = lax.axis_index("s")
    f0 = cid * FH
    bw0 = sid * BW

    pltpu.sync_copy(tab_hbm.at[cid], tab_v)
    pltpu.sync_copy(e0_hbm, e0_v)
    pltpu.sync_copy(hinv_hbm, hinv_v)

    f0v = _splat(f0)

    def pat(i, cr):
        col, row = cr
        fg = col + f0v
        sl = pl.ds(i * L, L)
        e0p[sl] = plsc.load_gather(e0_v, [fg])
        hip[sl] = plsc.load_gather(hinv_v, [fg])
        fbase[sl] = col * M
        rowp[sl] = row
        colg[sl] = fg
        ncol = col + L
        wrap = ncol >= FH
        ncol = jnp.where(wrap, ncol - FH, ncol)
        nrow = jnp.where(wrap, row + 1, row)
        return ncol, nrow

    col0 = lax.iota(jnp.int32, L)
    row0 = jnp.zeros((L,), jnp.int32)
    lax.fori_loop(0, NP // L, pat, (col0, row0))

    def chunk(c, _):
        b0 = bw0 + c * NB
        pltpu.sync_copy(x_hbm.at[pl.ds(b0, NB)], xb)

        def pre(i, _):
            sl = pl.ds(i * L, L)
            xv = plsc.load_gather(xb, [rowp[sl], colg[sl]])
            v = (xv - e0p[sl]) * hip[sl]
            bn = v.astype(jnp.int32)
            bn = jnp.minimum(jnp.maximum(bn, 0), M - 2)
            t = jnp.clip(v - bn.astype(jnp.float32), 0.0, 1.0)
            offb[sl] = fbase[sl] + bn
            tb[sl] = t
            return 0

        lax.fori_loop(0, NP // L, pre, 0)

        def lerp(i, _):
            sl = pl.ds(i * L, L)
            offl = offb[sl]
            offr = offl + 1
            tv = tb[sl]
            rv = rowp[sl]
            cv = colg[sl] - f0v
            dv = jnp.zeros((L,), jnp.int32)
            for d in range(D):
                lv = plsc.load_gather(tab_v, [offl, dv])
                rvv = plsc.load_gather(tab_v, [offr, dv])
                o = lv + tv * (rvv - lv)
                plsc.store_scatter(outb, [rv, cv, dv], o)
                dv = dv + 1
            return 0

        lax.fori_loop(0, NP // L, lerp, 0)

        pltpu.sync_copy(outb, out_hbm.at[pl.ds(b0, NB), cid])
        return 0

    lax.fori_loop(0, NCHUNK, chunk, 0)


@jax.jit
def kernel(x, bin_edges, boundary_embeddings):
    e0 = bin_edges[:, 0]
    h = bin_edges[:, 1] - bin_edges[:, 0]
    hinv = jnp.where(jnp.abs(h) < 1e-8, 1.0, 1.0 / h)
    tab3 = boundary_embeddings.reshape(N_CORES, FH * M, D)

    mesh = plsc.VectorSubcoreMesh(core_axis_name="c", subcore_axis_name="s")
    run = pl.kernel(
        _body,
        out_type=jax.ShapeDtypeStruct((B, N_CORES, FH, D), jnp.float32),
        mesh=mesh,
        compiler_params=pltpu.CompilerParams(
            use_tc_tiling_on_sc=False, needs_layout_passes=False),
        scratch_types=[
            pltpu.VMEM((FH * M, D), jnp.float32),
            pltpu.VMEM((NB, F), jnp.float32),
            pltpu.VMEM((F,), jnp.float32),
            pltpu.VMEM((F,), jnp.float32),
            pltpu.VMEM((NP,), jnp.float32),
            pltpu.VMEM((NP,), jnp.float32),
            pltpu.VMEM((NP,), jnp.int32),
            pltpu.VMEM((NP,), jnp.int32),
            pltpu.VMEM((NP,), jnp.int32),
            pltpu.VMEM((NP,), jnp.int32),
            pltpu.VMEM((NP,), jnp.float32),
            pltpu.VMEM((NB, FH, D), jnp.float32),
        ],
    )
    out = run(x, tab3, e0, hinv)
    return out.reshape(B, F, D)


# speedup vs baseline: 2.4882x; 2.4882x over previous
import jax
import jax.numpy as jnp
from jax import lax
from jax.experimental import pallas as pl
from jax.experimental.pallas import tpu as pltpu
from jax.experimental.pallas import tpu_sc as plsc

N_CORES = 2
N_SUBCORES = 16
L = 16

B = 16384
F = 100
M = 49
D = 32

FH = F // 2
BW = B // N_SUBCORES
NB = 8
NP = NB * FH
NCHUNK = BW // NB


def _splat(s):
    return lax.broadcast_in_dim(s, (L,), ())


def _body(x_hbm, tab_hbm, e0_hbm, hinv_hbm, out_hbm,
          tab_v, xb, e0_v, hinv_v, e0p, hip, fbase, rowp, colg, offb, tb, outb):
    cid = lax.axis_index("c")
    sid = lax.axis_index("s")
    f0 = cid * FH
    bw0 = sid * BW

    pltpu.sync_copy(tab_hbm.at[cid], tab_v)
    pltpu.sync_copy(e0_hbm, e0_v)
    pltpu.sync_copy(hinv_hbm, hinv_v)

    f0v = _splat(f0)

    def pat(i, cr):
        col, row = cr
        fg = col + f0v
        sl = pl.ds(i * L, L)
        e0p[sl] = plsc.load_gather(e0_v, [fg])
        hip[sl] = plsc.load_gather(hinv_v, [fg])
        fbase[sl] = col * M
        rowp[sl] = row
        colg[sl] = fg
        ncol = col + L
        wrap = ncol >= FH
        ncol = jnp.where(wrap, ncol - FH, ncol)
        nrow = jnp.where(wrap, row + 1, row)
        return ncol, nrow

    col0 = lax.iota(jnp.int32, L)
    row0 = jnp.zeros((L,), jnp.int32)
    lax.fori_loop(0, NP // L, pat, (col0, row0))

    def chunk(c, _):
        b0 = bw0 + c * NB
        pltpu.sync_copy(x_hbm.at[pl.ds(b0, NB)], xb)

        def pre(i, _):
            sl = pl.ds(i * L, L)
            xv = plsc.load_gather(xb, [rowp[sl], colg[sl]])
            v = (xv - e0p[sl]) * hip[sl]
            bn = v.astype(jnp.int32)
            bn = jnp.minimum(jnp.maximum(bn, 0), M - 2)
            t = jnp.clip(v - bn.astype(jnp.float32), 0.0, 1.0)
            offb[sl] = fbase[sl] + bn
            tb[sl] = t
            return 0

        lax.fori_loop(0, NP // L, pre, 0)

        def lerp(g, carry):
            r, cc = carry
            sl = pl.ds(g * L, L)
            ov = offb[sl]
            tvv = tb[sl]
            offs = [ov[k] for k in range(L)]
            ts = [tvv[k] for k in range(L)]
            for k in range(L):
                off = offs[k]
                tv = _splat(ts[k])
                l0 = tab_v[off, pl.ds(0, L)]
                l1 = tab_v[off, pl.ds(L, L)]
                r0 = tab_v[off + 1, pl.ds(0, L)]
                r1 = tab_v[off + 1, pl.ds(L, L)]
                outb[r, cc, pl.ds(0, L)] = l0 + tv * (r0 - l0)
                outb[r, cc, pl.ds(L, L)] = l1 + tv * (r1 - l1)
                nxt = cc + 1
                wrap = nxt >= FH
                cc = jnp.where(wrap, 0, nxt)
                r = jnp.where(wrap, r + 1, r)
            return r, cc

        lax.fori_loop(0, NP // L, lerp, (jnp.int32(0), jnp.int32(0)))

        pltpu.sync_copy(outb, out_hbm.at[pl.ds(b0, NB), cid])
        return 0

    lax.fori_loop(0, NCHUNK, chunk, 0)


@jax.jit
def kernel(x, bin_edges, boundary_embeddings):
    e0 = bin_edges[:, 0]
    h = bin_edges[:, 1] - bin_edges[:, 0]
    hinv = jnp.where(jnp.abs(h) < 1e-8, 1.0, 1.0 / h)
    tab3 = boundary_embeddings.reshape(N_CORES, FH * M, D)

    mesh = plsc.VectorSubcoreMesh(core_axis_name="c", subcore_axis_name="s")
    run = pl.kernel(
        _body,
        out_type=jax.ShapeDtypeStruct((B, N_CORES, FH, D), jnp.float32),
        mesh=mesh,
        compiler_params=pltpu.CompilerParams(
            use_tc_tiling_on_sc=False, needs_layout_passes=False),
        scratch_types=[
            pltpu.VMEM((FH * M, D), jnp.float32),
            pltpu.VMEM((NB, F), jnp.float32),
            pltpu.VMEM((F,), jnp.float32),
            pltpu.VMEM((F,), jnp.float32),
            pltpu.VMEM((NP,), jnp.float32),
            pltpu.VMEM((NP,), jnp.float32),
            pltpu.VMEM((NP,), jnp.int32),
            pltpu.VMEM((NP,), jnp.int32),
            pltpu.VMEM((NP,), jnp.int32),
            pltpu.VMEM((NP,), jnp.int32),
            pltpu.VMEM((NP,), jnp.float32),
            pltpu.VMEM((NB, FH, D), jnp.float32),
        ],
    )
    out = run(x, tab3, e0, hinv)
    return out.reshape(B, F, D)


# speedup vs baseline: 3.1578x; 1.2691x over previous
import jax
import jax.numpy as jnp
from jax import lax
from jax.experimental import pallas as pl
from jax.experimental.pallas import tpu as pltpu
from jax.experimental.pallas import tpu_sc as plsc

N_CORES = 2
N_SUBCORES = 16
L = 16

B = 16384
F = 100
M = 49
D = 32

FH = F // 2
BW = B // N_SUBCORES
NB = 8
NP = NB * FH
NCHUNK = BW // NB


def _splat(s):
    return lax.broadcast_in_dim(s, (L,), ())


def _body(x_hbm, tab_hbm, e0_hbm, hinv_hbm, out_hbm,
          tab_v, xb0, xb1, e0_v, hinv_v, e0p, hip, fbase, rowp, colg,
          offb, tb, outb0, outb1, sx0, sx1, so0, so1):
    cid = lax.axis_index("c")
    sid = lax.axis_index("s")
    f0 = cid * FH
    bw0 = sid * BW

    pltpu.sync_copy(tab_hbm.at[cid], tab_v)
    pltpu.sync_copy(e0_hbm, e0_v)
    pltpu.sync_copy(hinv_hbm, hinv_v)

    f0v = _splat(f0)

    def pat(i, cr):
        col, row = cr
        fg = col + f0v
        sl = pl.ds(i * L, L)
        e0p[sl] = plsc.load_gather(e0_v, [fg])
        hip[sl] = plsc.load_gather(hinv_v, [fg])
        fbase[sl] = col * M
        rowp[sl] = row
        colg[sl] = fg
        ncol = col + L
        wrap = ncol >= FH
        ncol = jnp.where(wrap, ncol - FH, ncol)
        nrow = jnp.where(wrap, row + 1, row)
        return ncol, nrow

    col0 = lax.iota(jnp.int32, L)
    row0 = jnp.zeros((L,), jnp.int32)
    lax.fori_loop(0, NP // L, pat, (col0, row0))

    pltpu.async_copy(x_hbm.at[pl.ds(bw0, NB)], xb0, sx0)

    bufs = ((xb0, sx0, outb0, so0, xb1, sx1),
            (xb1, sx1, outb1, so1, xb0, sx0))

    def chunk2(ci, _):
        for j, (xb, sx, ob, so, nxb, nsx) in enumerate(bufs):
            c = ci * 2 + j
            b0 = bw0 + c * NB

            cn = c + 1
            cn = jnp.where(cn >= NCHUNK, 0, cn)
            pltpu.async_copy(x_hbm.at[pl.ds(bw0 + cn * NB, NB)], nxb, nsx)
            pltpu.make_async_copy(x_hbm.at[pl.ds(b0, NB)], xb, sx).wait()

            @plsc.parallel_loop(0, NP // L, unroll=2)
            def pre(i):
                sl = pl.ds(i * L, L)
                xv = plsc.load_gather(xb, [rowp[sl], colg[sl]])
                v = (xv - e0p[sl]) * hip[sl]
                bn = v.astype(jnp.int32)
                bn = jnp.minimum(jnp.maximum(bn, 0), M - 2)
                t = jnp.clip(v - bn.astype(jnp.float32), 0.0, 1.0)
                offb[sl] = fbase[sl] + bn
                tb[sl] = t

            @pl.when(c >= 2)
            def _():
                pltpu.make_async_copy(ob, out_hbm.at[pl.ds(b0, NB), cid],
                                      so).wait()

            @plsc.parallel_loop(0, NP // L, unroll=2,
                                carry=(jnp.int32(0), jnp.int32(0)))
            def lerp(g, carry):
                r, cc = carry
                sl = pl.ds(g * L, L)
                ov = offb[sl]
                tvv = tb[sl]
                for k in range(L):
                    off = ov[k]
                    tv = _splat(tvv[k])
                    l0 = tab_v[off, pl.ds(0, L)]
                    l1 = tab_v[off, pl.ds(L, L)]
                    r0 = tab_v[off + 1, pl.ds(0, L)]
                    r1 = tab_v[off + 1, pl.ds(L, L)]
                    ob[r, cc, pl.ds(0, L)] = l0 + tv * (r0 - l0)
                    ob[r, cc, pl.ds(L, L)] = l1 + tv * (r1 - l1)
                    nxt = cc + 1
                    wrap = nxt >= FH
                    cc = jnp.where(wrap, 0, nxt)
                    r = jnp.where(wrap, r + 1, r)
                return r, cc

            pltpu.async_copy(ob, out_hbm.at[pl.ds(b0, NB), cid], so)
        return 0

    lax.fori_loop(0, NCHUNK // 2, chunk2, 0)

    pltpu.make_async_copy(x_hbm.at[pl.ds(bw0, NB)], xb0, sx0).wait()
    pltpu.make_async_copy(outb0, out_hbm.at[pl.ds(bw0, NB), cid], so0).wait()
    pltpu.make_async_copy(outb1, out_hbm.at[pl.ds(bw0, NB), cid], so1).wait()


@jax.jit
def kernel(x, bin_edges, boundary_embeddings):
    e0 = bin_edges[:, 0]
    h = bin_edges[:, 1] - bin_edges[:, 0]
    hinv = jnp.where(jnp.abs(h) < 1e-8, 1.0, 1.0 / h)
    tab3 = boundary_embeddings.reshape(N_CORES, FH * M, D)

    mesh = plsc.VectorSubcoreMesh(core_axis_name="c", subcore_axis_name="s")
    run = pl.kernel(
        _body,
        out_type=jax.ShapeDtypeStruct((B, N_CORES, FH, D), jnp.float32),
        mesh=mesh,
        compiler_params=pltpu.CompilerParams(
            use_tc_tiling_on_sc=False, needs_layout_passes=False),
        scratch_types=[
            pltpu.VMEM((FH * M, D), jnp.float32),
            pltpu.VMEM((NB, F), jnp.float32),
            pltpu.VMEM((NB, F), jnp.float32),
            pltpu.VMEM((F,), jnp.float32),
            pltpu.VMEM((F,), jnp.float32),
            pltpu.VMEM((NP,), jnp.float32),
            pltpu.VMEM((NP,), jnp.float32),
            pltpu.VMEM((NP,), jnp.int32),
            pltpu.VMEM((NP,), jnp.int32),
            pltpu.VMEM((NP,), jnp.int32),
            pltpu.VMEM((NP,), jnp.int32),
            pltpu.VMEM((NP,), jnp.float32),
            pltpu.VMEM((NB, FH, D), jnp.float32),
            pltpu.VMEM((NB, FH, D), jnp.float32),
            pltpu.SemaphoreType.DMA,
            pltpu.SemaphoreType.DMA,
            pltpu.SemaphoreType.DMA,
            pltpu.SemaphoreType.DMA,
        ],
    )
    out = run(x, tab3, e0, hinv)
    return out.reshape(B, F, D)


# speedup vs baseline: 7.5411x; 2.3881x over previous
import jax
import jax.numpy as jnp
from jax import lax
from jax.experimental import pallas as pl
from jax.experimental.pallas import tpu as pltpu
from jax.experimental.pallas import tpu_sc as plsc

N_CORES = 2
N_SUBCORES = 16
L = 16

B = 16384
F = 100
M = 49
D = 32

FH = F // 2
MP = M * FH
RW = D + 1
BT = B // 128
BTW = BT // N_SUBCORES
DT = D // 8


def _splat(s):
    return lax.broadcast_in_dim(s, (L,), ())


def _body(x_hbm, tab_hbm, e0_hbm, hinv_hbm, out_hbm,
          tab_v, xb, e0_v, hinv_v, sb0, sb1, so0, so1):
    cid = lax.axis_index("c")
    sid = lax.axis_index("s")
    f0 = cid * FH
    bw0 = sid * (BTW * 128)
    bt0 = sid * BTW

    pltpu.sync_copy(tab_hbm.at[cid], tab_v)
    pltpu.sync_copy(e0_hbm, e0_v)
    pltpu.sync_copy(hinv_hbm, hinv_v)

    iot = lax.iota(jnp.int32, L)
    zero16 = jnp.zeros((L,), jnp.int32)

    def btloop(bt, _):
        pltpu.sync_copy(x_hbm.at[pl.ds(bw0 + bt * 128, 128)], xb)

        def floop2(fi, _):
            for j, (sb, so) in enumerate(((sb0, so0), (sb1, so1))):
                fl = fi * 2 + j
                fg = fl + f0
                fgv = _splat(fg)
                e0f = plsc.load_gather(e0_v, [fgv])
                hif = plsc.load_gather(hinv_v, [fgv])
                rowbase = _splat(fl * M)
                dst = out_hbm.at[cid, fl, :, pl.ds(bt0 + bt, 1)]

                @pl.when(jnp.logical_or(fl >= 2, bt > 0))
                def _():
                    pltpu.make_async_copy(sb, dst, so).wait()

                for g in range(128 // L):
                    xv = plsc.load_gather(xb, [iot + g * L, fgv])
                    v = (xv - e0f) * hif
                    bn = v.astype(jnp.int32)
                    bn = jnp.minimum(jnp.maximum(bn, 0), M - 2)
                    tv = jnp.clip(v - bn.astype(jnp.float32), 0.0, 1.0)
                    rl = bn + rowbase
                    rr = rl + 1

                    @plsc.parallel_loop(0, D, unroll=4,
                                        carry=(zero16, jnp.int32(0),
                                               jnp.int32(0)))
                    def dloop(d, carry, rl=rl, rr=rr, tv=tv, sb=sb, g=g):
                        dv, dhi, dlo = carry
                        lv = plsc.load_gather(tab_v, [rl, dv])
                        rv = plsc.load_gather(tab_v, [rr, dv])
                        sb[dhi, 0, dlo, pl.ds(g * L, L)] = lv + tv * (rv - lv)
                        nlo = dlo + 1
                        wrap = nlo >= 8
                        nlo = jnp.where(wrap, 0, nlo)
                        nhi = jnp.where(wrap, dhi + 1, dhi)
                        return dv + 1, nhi, nlo

                pltpu.async_copy(sb, dst, so)
            return 0

        lax.fori_loop(0, FH // 2, floop2, 0)
        return 0

    lax.fori_loop(0, BTW, btloop, 0)

    last = out_hbm.at[cid, 0, :, pl.ds(bt0, 1)]
    pltpu.make_async_copy(sb0, last, so0).wait()
    pltpu.make_async_copy(sb1, last, so1).wait()


@jax.jit
def kernel(x, bin_edges, boundary_embeddings):
    e0 = bin_edges[:, 0]
    h = bin_edges[:, 1] - bin_edges[:, 0]
    hinv = jnp.where(jnp.abs(h) < 1e-8, 1.0, 1.0 / h)
    tabp = jnp.pad(boundary_embeddings.reshape(F * M, D),
                   ((0, 0), (0, RW - D))).reshape(N_CORES, MP, RW)

    mesh = plsc.VectorSubcoreMesh(core_axis_name="c", subcore_axis_name="s")
    run = pl.kernel(
        _body,
        out_type=jax.ShapeDtypeStruct((N_CORES, FH, DT, BT, 8, 128),
                                      jnp.float32),
        mesh=mesh,
        compiler_params=pltpu.CompilerParams(
            use_tc_tiling_on_sc=False, needs_layout_passes=False),
        scratch_types=[
            pltpu.VMEM((MP, RW), jnp.float32),
            pltpu.VMEM((128, F), jnp.float32),
            pltpu.VMEM((F,), jnp.float32),
            pltpu.VMEM((F,), jnp.float32),
            pltpu.VMEM((DT, 1, 8, 128), jnp.float32),
            pltpu.VMEM((DT, 1, 8, 128), jnp.float32),
            pltpu.SemaphoreType.DMA,
            pltpu.SemaphoreType.DMA,
        ],
    )
    phys = run(x, tabp, e0, hinv)
    p6 = phys.reshape(F, DT, BT, 8, 128)
    return p6.transpose(2, 4, 0, 1, 3).reshape(B, F, D)


# speedup vs baseline: 13.1794x; 1.7477x over previous
import jax
import jax.numpy as jnp
from jax import lax
from jax.experimental import pallas as pl
from jax.experimental.pallas import tpu as pltpu
from jax.experimental.pallas import tpu_sc as plsc

N_CORES = 2
N_SUBCORES = 16
L = 16

B = 16384
F = 100
M = 49
D = 32

FH = F // 2
MP = M * FH
RW = D // 2 + 1
BT = B // 128
BTW = BT // N_SUBCORES
DT = D // 8


def _splat(s):
    return lax.broadcast_in_dim(s, (L,), ())


def _body(x_hbm, tab_hbm, e0_hbm, hinv_hbm, out_hbm,
          tab_v, xb, e0_v, hinv_v, sb0, sb1, so0, so1):
    cid = lax.axis_index("c")
    sid = lax.axis_index("s")
    f0 = cid * FH
    bw0 = sid * (BTW * 128)
    bt0 = sid * BTW

    pltpu.sync_copy(tab_hbm.at[cid], tab_v)
    pltpu.sync_copy(e0_hbm, e0_v)
    pltpu.sync_copy(hinv_hbm, hinv_v)

    iot = lax.iota(jnp.int32, L)
    zero16 = jnp.zeros((L,), jnp.int32)

    def btloop(bt, _):
        pltpu.sync_copy(x_hbm.at[pl.ds(bw0 + bt * 128, 128)], xb)

        def floop2(fi, _):
            for j, (sb, so) in enumerate(((sb0, so0), (sb1, so1))):
                fl = fi * 2 + j
                fg = fl + f0
                fgv = _splat(fg)
                e0f = plsc.load_gather(e0_v, [fgv])
                hif = plsc.load_gather(hinv_v, [fgv])
                rowbase = _splat(fl * M)
                dst = out_hbm.at[cid, fl, :, pl.ds(bt0 + bt, 1)]

                @pl.when(jnp.logical_or(fl >= 2, bt > 0))
                def _():
                    pltpu.make_async_copy(sb, dst, so).wait()

                for g in range(128 // L):
                    xv = plsc.load_gather(xb, [iot + g * L, fgv])
                    v = (xv - e0f) * hif
                    bn = v.astype(jnp.int32)
                    bn = jnp.minimum(jnp.maximum(bn, 0), M - 2)
                    tv = jnp.clip(v - bn.astype(jnp.float32), 0.0, 1.0)
                    rl = bn + rowbase
                    rr = rl + 1

                    tp = plsc.pack(tv, tv, format=plsc.PackFormat.INTERLEAVED)

                    @plsc.parallel_loop(0, D // 2, unroll=4,
                                        carry=(zero16, jnp.int32(0),
                                               jnp.int32(0)))
                    def dloop(d, carry, rl=rl, rr=rr, tp=tp, sb=sb, g=g):
                        dv, dhi, dlo = carry
                        lv = plsc.bitcast(
                            plsc.load_gather(tab_v, [rl, dv]), jnp.bfloat16)
                        rv = plsc.bitcast(
                            plsc.load_gather(tab_v, [rr, dv]), jnp.bfloat16)
                        o = lv + tp * (rv - lv)
                        oa, ob = plsc.unpack(
                            o, format=plsc.PackFormat.INTERLEAVED)
                        sb[dhi, 0, dlo, pl.ds(g * L, L)] = oa
                        sb[dhi, 0, dlo + 1, pl.ds(g * L, L)] = ob
                        nlo = dlo + 2
                        wrap = nlo >= 8
                        nlo = jnp.where(wrap, 0, nlo)
                        nhi = jnp.where(wrap, dhi + 1, dhi)
                        return dv + 1, nhi, nlo

                pltpu.async_copy(sb, dst, so)
            return 0

        lax.fori_loop(0, FH // 2, floop2, 0)
        return 0

    lax.fori_loop(0, BTW, btloop, 0)

    last = out_hbm.at[cid, 0, :, pl.ds(bt0, 1)]
    pltpu.make_async_copy(sb0, last, so0).wait()
    pltpu.make_async_copy(sb1, last, so1).wait()


@jax.jit
def kernel(x, bin_edges, boundary_embeddings):
    e0 = bin_edges[:, 0]
    h = bin_edges[:, 1] - bin_edges[:, 0]
    hinv = jnp.where(jnp.abs(h) < 1e-8, 1.0, 1.0 / h)
    emb_bf = boundary_embeddings.astype(jnp.bfloat16).reshape(F * M, D // 2, 2)
    tab_i = jax.lax.bitcast_convert_type(emb_bf, jnp.int32)
    tabp = jnp.pad(tab_i, ((0, 0), (0, 1))).reshape(N_CORES, MP, RW)

    mesh = plsc.VectorSubcoreMesh(core_axis_name="c", subcore_axis_name="s")
    run = pl.kernel(
        _body,
        out_type=jax.ShapeDtypeStruct((N_CORES, FH, DT, BT, 8, 128),
                                      jnp.float32),
        mesh=mesh,
        compiler_params=pltpu.CompilerParams(
            use_tc_tiling_on_sc=False, needs_layout_passes=False),
        scratch_types=[
            pltpu.VMEM((MP, RW), jnp.int32),
            pltpu.VMEM((128, F), jnp.float32),
            pltpu.VMEM((F,), jnp.float32),
            pltpu.VMEM((F,), jnp.float32),
            pltpu.VMEM((DT, 1, 8, 128), jnp.float32),
            pltpu.VMEM((DT, 1, 8, 128), jnp.float32),
            pltpu.SemaphoreType.DMA,
            pltpu.SemaphoreType.DMA,
        ],
    )
    phys = run(x, tabp, e0, hinv)
    p6 = phys.reshape(F, DT, BT, 8, 128)
    return p6.transpose(2, 4, 0, 1, 3).reshape(B, F, D)
